# baseline (speedup 1.0000x reference)
import functools

import jax
import jax.numpy as jnp
from jax import lax
from jax.experimental import pallas as pl
from jax.experimental.pallas import tpu as pltpu
from jax.experimental.pallas import tpu_sc as plsc

_N = 10000
_NP = 10112
_NC, _NS = 2, 16
_NW = _NC * _NS
_RPS = _NP // _NS
_EPAD = 163840
_EPW = _EPAD // _NW
_BLK = 128
_NBLK = _EPW // _BLK
_BI = 400

_mesh = plsc.VectorSubcoreMesh(core_axis_name="c", subcore_axis_name="s",
                               num_cores=_NC, num_subcores=_NS)
_sc_params = pltpu.CompilerParams(use_tc_tiling_on_sc=False,
                                  needs_layout_passes=False)


def _zero_rows(ref, d):
    def zrow(i, carry):
        for j in range(d // 16):
            ref[i, pl.ds(j * 16, 16)] = jnp.zeros((16,), jnp.float32)
        return carry
    lax.fori_loop(0, _RPS, zrow, 0)


def _make_gather_scatter(d, stage_table):
    scratch = [
        pltpu.VMEM((_EPW,), jnp.int32),
        pltpu.VMEM((_EPW,), jnp.int32),
        pltpu.VMEM((_BLK,), jnp.int32),
        pltpu.VMEM((_BLK, d), jnp.float32),
        pltpu.VMEM((_BLK, d), jnp.float32),
        pltpu.VMEM((_RPS, d), jnp.float32),
        pltpu.VMEM_SHARED((_NP, d), jnp.float32),
        pltpu.SemaphoreType.DMA,
        pltpu.SemaphoreType.DMA,
    ]
    if stage_table:
        scratch.insert(7, pltpu.VMEM_SHARED((_NP, d), jnp.float32))

    @functools.partial(
        pl.kernel,
        out_type=jax.ShapeDtypeStruct((_NC, _NP, d), jnp.float32),
        mesh=_mesh,
        compiler_params=_sc_params,
        scratch_types=scratch,
    )
    def scat(table, srcs, dsts, out, src_v, dst_v, dst_cur, rows0, rows1,
             stage_v, acc, *rest):
        tbl_s = rest[0] if stage_table else None
        sem0, sem1 = rest[-2], rest[-1]
        cid = lax.axis_index("c")
        sid = lax.axis_index("s")
        wid = sid * _NC + cid
        base = wid * _EPW
        pltpu.sync_copy(srcs.at[pl.ds(base, _EPW)], src_v)
        pltpu.sync_copy(dsts.at[pl.ds(base, _EPW)], dst_v)
        if stage_table:
            pltpu.sync_copy(table.at[pl.ds(sid * _RPS, _RPS)], stage_v)
            pltpu.sync_copy(stage_v, tbl_s.at[pl.ds(sid * _RPS, _RPS)])
        _zero_rows(stage_v, d)
        pltpu.sync_copy(stage_v, acc.at[pl.ds(sid * _RPS, _RPS)])
        plsc.subcore_barrier()

        gsrc = tbl_s if stage_table else table

        def gather(blk, rbuf, sem):
            pltpu.async_copy(gsrc.at[src_v.at[pl.ds(blk * _BLK, _BLK)]],
                             rbuf, sem)

        def scatter(blk, rbuf):
            off = blk * _BLK
            for j in range(_BLK // 16):
                dst_cur[pl.ds(j * 16, 16)] = dst_v[pl.ds(off + j * 16, 16)]
            pltpu.sync_copy(rbuf, acc.at[dst_cur], add=True)

        gather(0, rows0, sem0)

        def step(i, carry):
            b0 = 2 * i
            pltpu.make_async_copy(table.at[src_v.at[pl.ds(0, _BLK)]], rows0,
                                  sem0).wait()
            gather(b0 + 1, rows1, sem1)
            scatter(b0, rows0)
            pltpu.make_async_copy(table.at[src_v.at[pl.ds(0, _BLK)]], rows1,
                                  sem1).wait()
            gather(jnp.minimum(b0 + 2, _NBLK - 1), rows0, sem0)
            scatter(b0 + 1, rows1)
            return carry

        lax.fori_loop(0, _NBLK // 2, step, 0)
        pltpu.make_async_copy(table.at[src_v.at[pl.ds(0, _BLK)]], rows0,
                              sem0).wait()
        plsc.subcore_barrier()
        pltpu.sync_copy(acc.at[pl.ds(sid * _RPS, _RPS)], stage_v)
        pltpu.sync_copy(stage_v, out.at[cid, pl.ds(sid * _RPS, _RPS)])

    return scat


@functools.partial(
    pl.kernel,
    out_type=jax.ShapeDtypeStruct((_NW, _NP), jnp.float32),
    mesh=_mesh,
    compiler_params=_sc_params,
    scratch_types=[
        pltpu.VMEM((_EPW,), jnp.int32),
        pltpu.VMEM((_NP,), jnp.float32),
    ],
)
def _deg_kernel(dsts, out, dst_v, deg_v):
    cid = lax.axis_index("c")
    sid = lax.axis_index("s")
    wid = sid * _NC + cid
    pltpu.sync_copy(dsts.at[pl.ds(wid * _EPW, _EPW)], dst_v)

    def zero(i, carry):
        deg_v[pl.ds(i * 16, 16)] = jnp.zeros((16,), jnp.float32)
        return carry
    lax.fori_loop(0, _NP // 16, zero, 0)

    ones = jnp.full((16,), 1.0, jnp.float32)

    def step(i, carry):
        idx = dst_v[pl.ds(i * 16, 16)]
        plsc.addupdate_scatter(deg_v, [idx], ones)
        return carry

    lax.fori_loop(0, _EPW // 16, step, 0)
    pltpu.sync_copy(deg_v, out.at[wid])


_scat16 = _make_gather_scatter(16, stage_table=True)

_EPT = _EPAD // _NS
_NBLKC = _EPT // _BLK


@functools.partial(
    pl.kernel,
    out_type=jax.ShapeDtypeStruct((_NC, _NP, 32), jnp.float32),
    mesh=_mesh,
    compiler_params=_sc_params,
    scratch_types=[
        pltpu.VMEM((_EPT,), jnp.int32),
        pltpu.VMEM((_EPT,), jnp.int32),
        pltpu.VMEM((_BLK,), jnp.int32),
        pltpu.VMEM((_BLK, 32), jnp.float32),
        pltpu.VMEM((_BLK, 32), jnp.float32),
        pltpu.VMEM((_RPS, 32), jnp.float32),
        pltpu.VMEM_SHARED((_NP, 32), jnp.float32),
        pltpu.VMEM_SHARED((_NP, 32), jnp.float32),
        pltpu.SemaphoreType.DMA,
        pltpu.SemaphoreType.DMA,
    ],
)
def _scat64_colsplit(table, srcs, dsts, out, src_v, dst_v, dst_cur, rows0,
                     rows1, stage_v, acc, tbl_s, sem0, sem1):
    cid = lax.axis_index("c")
    sid = lax.axis_index("s")
    base = sid * _EPT
    pltpu.sync_copy(srcs.at[pl.ds(base, _EPT)], src_v)
    pltpu.sync_copy(dsts.at[pl.ds(base, _EPT)], dst_v)
    pltpu.sync_copy(
        table.at[pl.ds(sid * _RPS, _RPS), pl.ds(cid * 32, 32)], stage_v)
    pltpu.sync_copy(stage_v, tbl_s.at[pl.ds(sid * _RPS, _RPS)])
    _zero_rows(stage_v, 32)
    pltpu.sync_copy(stage_v, acc.at[pl.ds(sid * _RPS, _RPS)])
    plsc.subcore_barrier()

    def gather(blk, rbuf, sem):
        pltpu.async_copy(tbl_s.at[src_v.at[pl.ds(blk * _BLK, _BLK)]],
                         rbuf, sem)

    def scatter(blk, rbuf):
        off = blk * _BLK
        for j in range(_BLK // 16):
            dst_cur[pl.ds(j * 16, 16)] = dst_v[pl.ds(off + j * 16, 16)]
        pltpu.sync_copy(rbuf, acc.at[dst_cur], add=True)

    gather(0, rows0, sem0)

    def step(i, carry):
        b0 = 2 * i
        pltpu.make_async_copy(table.at[pl.ds(0, _BLK), pl.ds(0, 32)], rows0,
                              sem0).wait()
        gather(b0 + 1, rows1, sem1)
        scatter(b0, rows0)
        pltpu.make_async_copy(table.at[pl.ds(0, _BLK), pl.ds(0, 32)], rows1,
                              sem1).wait()
        gather(jnp.minimum(b0 + 2, _NBLKC - 1), rows0, sem0)
        scatter(b0 + 1, rows1)
        return carry

    lax.fori_loop(0, _NBLKC // 2, step, 0)
    pltpu.make_async_copy(table.at[pl.ds(0, _BLK), pl.ds(0, 32)], rows0,
                          sem0).wait()
    plsc.subcore_barrier()
    pltpu.sync_copy(acc.at[pl.ds(sid * _RPS, _RPS)], stage_v)
    pltpu.sync_copy(stage_v, out.at[cid, pl.ds(sid * _RPS, _RPS)])


def _stage1_body(x_ref, w_ref, dp_ref, hs_ref, dis_ref):
    ones = jnp.ones((_NW, 1), jnp.float32)
    deg = lax.dot_general(dp_ref[...], ones, (((0,), (0,)), ((), ())),
                          preferred_element_type=jnp.float32)
    dis = lax.rsqrt(jnp.maximum(deg, 1.0))
    h = jnp.dot(x_ref[...], w_ref[...], preferred_element_type=jnp.float32)
    hs_ref[...] = h * dis
    dis_ref[...] = dis


def _stage3_body(a_ref, dis_ref, b1_ref, w_ref, hs_ref):
    dis = dis_ref[...]
    agg = jnp.concatenate([a_ref[0], a_ref[1]], axis=1)
    a = agg * dis + b1_ref[...]
    hidden = jnp.maximum(a, 0.0)
    h2 = jnp.dot(hidden, w_ref[...], preferred_element_type=jnp.float32)
    hs_ref[...] = h2 * dis


def _stage5_body(a_ref, dis_ref, b2_ref, z_ref):
    z_ref[...] = (a_ref[0] + a_ref[1]) * dis_ref[...] + b2_ref[...]


def _decoder_body(zb_ref, za_ref, out_ref):
    acc = lax.dot_general(zb_ref[...], za_ref[...], (((1,), (1,)), ((), ())),
                          preferred_element_type=jnp.float32)
    out_ref[...] = jax.nn.sigmoid(acc)


def _decoder(z):
    n, d = z.shape
    return pl.pallas_call(
        _decoder_body,
        grid=(n // _BI,),
        in_specs=[
            pl.BlockSpec((_BI, d), lambda i: (i, 0)),
            pl.BlockSpec((n, d), lambda i: (0, 0)),
        ],
        out_specs=pl.BlockSpec((_BI, n), lambda i: (i, 0)),
        out_shape=jax.ShapeDtypeStruct((n, n), jnp.float32),
    )(z, z)


def kernel(X, edge_index, W1, b1, W2, b2):
    E = edge_index.shape[1]
    pad = jnp.full((_EPAD - E,), _N, jnp.int32)
    srcp = jnp.concatenate([edge_index[0], pad])
    dstp = jnp.concatenate([edge_index[1], pad])
    Xp = jnp.pad(X, ((0, _NP - _N), (0, 0)))

    degacc = _deg_kernel(dstp)

    hs1, dis = pl.pallas_call(
        _stage1_body,
        out_shape=[jax.ShapeDtypeStruct((_NP, 64), jnp.float32),
                   jax.ShapeDtypeStruct((_NP, 1), jnp.float32)],
    )(Xp, W1, degacc)

    agg1 = _scat64_colsplit(hs1, srcp, dstp)

    hs2 = pl.pallas_call(
        _stage3_body,
        out_shape=jax.ShapeDtypeStruct((_NP, 16), jnp.float32),
    )(agg1, dis, b1.reshape(1, 64), W2)

    agg2 = _scat16(hs2, srcp, dstp)

    z_pad = pl.pallas_call(
        _stage5_body,
        out_shape=jax.ShapeDtypeStruct((_NP, 16), jnp.float32),
    )(agg2, dis, b2.reshape(1, 16))

    z = z_pad[:_N]
    adj = _decoder(z)
    return (adj, z, z, z)


# speedup vs baseline: 1.0333x; 1.0333x over previous
import functools

import jax
import jax.numpy as jnp
from jax import lax
from jax.experimental import pallas as pl
from jax.experimental.pallas import tpu as pltpu
from jax.experimental.pallas import tpu_sc as plsc

_N = 10000
_NP = 10112
_NC, _NS = 2, 16
_NW = _NC * _NS
_RPS = _NP // _NS
_EPAD = 163840
_EPW = _EPAD // _NW
_BLK = 128
_NBLK = _EPW // _BLK
_BI = 400

_mesh = plsc.VectorSubcoreMesh(core_axis_name="c", subcore_axis_name="s",
                               num_cores=_NC, num_subcores=_NS)
_sc_params = pltpu.CompilerParams(use_tc_tiling_on_sc=False,
                                  needs_layout_passes=False)


def _zero_rows(ref, d):
    def zrow(i, carry):
        for j in range(d // 16):
            ref[i, pl.ds(j * 16, 16)] = jnp.zeros((16,), jnp.float32)
        return carry
    lax.fori_loop(0, _RPS, zrow, 0)


def _make_gather_scatter(d, stage_table):
    scratch = [
        pltpu.VMEM((_EPW,), jnp.int32),
        pltpu.VMEM((_EPW,), jnp.int32),
        pltpu.VMEM((_BLK,), jnp.int32),
        pltpu.VMEM((_BLK, d), jnp.float32),
        pltpu.VMEM((_BLK, d), jnp.float32),
        pltpu.VMEM((_RPS, d), jnp.float32),
        pltpu.VMEM_SHARED((_NP, d), jnp.float32),
        pltpu.SemaphoreType.DMA,
        pltpu.SemaphoreType.DMA,
    ]
    if stage_table:
        scratch.insert(7, pltpu.VMEM_SHARED((_NP, d), jnp.float32))

    @functools.partial(
        pl.kernel,
        out_type=jax.ShapeDtypeStruct((_NC, _NP, d), jnp.float32),
        mesh=_mesh,
        compiler_params=_sc_params,
        scratch_types=scratch,
    )
    def scat(table, srcs, dsts, out, src_v, dst_v, dst_cur, rows0, rows1,
             stage_v, acc, *rest):
        tbl_s = rest[0] if stage_table else None
        sem0, sem1 = rest[-2], rest[-1]
        cid = lax.axis_index("c")
        sid = lax.axis_index("s")
        wid = sid * _NC + cid
        base = wid * _EPW
        pltpu.sync_copy(srcs.at[pl.ds(base, _EPW)], src_v)
        pltpu.sync_copy(dsts.at[pl.ds(base, _EPW)], dst_v)
        if stage_table:
            pltpu.sync_copy(table.at[pl.ds(sid * _RPS, _RPS)], stage_v)
            pltpu.sync_copy(stage_v, tbl_s.at[pl.ds(sid * _RPS, _RPS)])
        _zero_rows(stage_v, d)
        pltpu.sync_copy(stage_v, acc.at[pl.ds(sid * _RPS, _RPS)])
        plsc.subcore_barrier()

        gsrc = tbl_s if stage_table else table

        def gather(blk, rbuf, sem):
            pltpu.async_copy(gsrc.at[src_v.at[pl.ds(blk * _BLK, _BLK)]],
                             rbuf, sem)

        def scatter(blk, rbuf):
            off = blk * _BLK
            for j in range(_BLK // 16):
                dst_cur[pl.ds(j * 16, 16)] = dst_v[pl.ds(off + j * 16, 16)]
            pltpu.sync_copy(rbuf, acc.at[dst_cur], add=True)

        gather(0, rows0, sem0)

        def step(i, carry):
            b0 = 2 * i
            pltpu.make_async_copy(table.at[src_v.at[pl.ds(0, _BLK)]], rows0,
                                  sem0).wait()
            gather(b0 + 1, rows1, sem1)
            scatter(b0, rows0)
            pltpu.make_async_copy(table.at[src_v.at[pl.ds(0, _BLK)]], rows1,
                                  sem1).wait()
            gather(jnp.minimum(b0 + 2, _NBLK - 1), rows0, sem0)
            scatter(b0 + 1, rows1)
            return carry

        lax.fori_loop(0, _NBLK // 2, step, 0)
        pltpu.make_async_copy(table.at[src_v.at[pl.ds(0, _BLK)]], rows0,
                              sem0).wait()
        plsc.subcore_barrier()
        pltpu.sync_copy(acc.at[pl.ds(sid * _RPS, _RPS)], stage_v)
        pltpu.sync_copy(stage_v, out.at[cid, pl.ds(sid * _RPS, _RPS)])

    return scat


@functools.partial(
    pl.kernel,
    out_type=jax.ShapeDtypeStruct((_NW, _NP), jnp.float32),
    mesh=_mesh,
    compiler_params=_sc_params,
    scratch_types=[
        pltpu.VMEM((_EPW,), jnp.int32),
        pltpu.VMEM((_NP,), jnp.float32),
    ],
)
def _deg_kernel(dsts, out, dst_v, deg_v):
    cid = lax.axis_index("c")
    sid = lax.axis_index("s")
    wid = sid * _NC + cid
    pltpu.sync_copy(dsts.at[pl.ds(wid * _EPW, _EPW)], dst_v)

    def zero(i, carry):
        deg_v[pl.ds(i * 16, 16)] = jnp.zeros((16,), jnp.float32)
        return carry
    lax.fori_loop(0, _NP // 16, zero, 0)

    ones = jnp.full((16,), 1.0, jnp.float32)

    def step(i, carry):
        idx = dst_v[pl.ds(i * 16, 16)]
        plsc.addupdate_scatter(deg_v, [idx], ones)
        return carry

    lax.fori_loop(0, _EPW // 16, step, 0)
    pltpu.sync_copy(deg_v, out.at[wid])


_scat16 = _make_gather_scatter(16, stage_table=True)

_EPT = _EPAD // _NS
_NBLKC = _EPT // _BLK


@functools.partial(
    pl.kernel,
    out_type=jax.ShapeDtypeStruct((_NC, _NP, 32), jnp.float32),
    mesh=_mesh,
    compiler_params=_sc_params,
    scratch_types=[
        pltpu.VMEM((_EPT,), jnp.int32),
        pltpu.VMEM((_EPT,), jnp.int32),
        pltpu.VMEM((_BLK,), jnp.int32),
        pltpu.VMEM((_BLK, 32), jnp.float32),
        pltpu.VMEM((_BLK, 32), jnp.float32),
        pltpu.VMEM((_RPS, 32), jnp.float32),
        pltpu.VMEM_SHARED((_NP, 32), jnp.float32),
        pltpu.VMEM_SHARED((_NP, 32), jnp.float32),
        pltpu.SemaphoreType.DMA,
        pltpu.SemaphoreType.DMA,
    ],
)
def _scat64_colsplit(table, srcs, dsts, out, src_v, dst_v, dst_cur, rows0,
                     rows1, stage_v, acc, tbl_s, sem0, sem1):
    cid = lax.axis_index("c")
    sid = lax.axis_index("s")
    base = sid * _EPT
    pltpu.sync_copy(srcs.at[pl.ds(base, _EPT)], src_v)
    pltpu.sync_copy(dsts.at[pl.ds(base, _EPT)], dst_v)
    pltpu.sync_copy(
        table.at[pl.ds(sid * _RPS, _RPS), pl.ds(cid * 32, 32)], stage_v)
    pltpu.sync_copy(stage_v, tbl_s.at[pl.ds(sid * _RPS, _RPS)])
    _zero_rows(stage_v, 32)
    pltpu.sync_copy(stage_v, acc.at[pl.ds(sid * _RPS, _RPS)])
    plsc.subcore_barrier()

    def gather(blk, rbuf, sem):
        pltpu.async_copy(tbl_s.at[src_v.at[pl.ds(blk * _BLK, _BLK)]],
                         rbuf, sem)

    def scatter(blk, rbuf):
        off = blk * _BLK
        for j in range(_BLK // 16):
            dst_cur[pl.ds(j * 16, 16)] = dst_v[pl.ds(off + j * 16, 16)]
        pltpu.sync_copy(rbuf, acc.at[dst_cur], add=True)

    gather(0, rows0, sem0)

    def step(i, carry):
        b0 = 2 * i
        pltpu.make_async_copy(table.at[pl.ds(0, _BLK), pl.ds(0, 32)], rows0,
                              sem0).wait()
        gather(b0 + 1, rows1, sem1)
        scatter(b0, rows0)
        pltpu.make_async_copy(table.at[pl.ds(0, _BLK), pl.ds(0, 32)], rows1,
                              sem1).wait()
        gather(jnp.minimum(b0 + 2, _NBLKC - 1), rows0, sem0)
        scatter(b0 + 1, rows1)
        return carry

    lax.fori_loop(0, _NBLKC // 2, step, 0)
    pltpu.make_async_copy(table.at[pl.ds(0, _BLK), pl.ds(0, 32)], rows0,
                          sem0).wait()
    plsc.subcore_barrier()
    pltpu.sync_copy(acc.at[pl.ds(sid * _RPS, _RPS)], stage_v)
    pltpu.sync_copy(stage_v, out.at[cid, pl.ds(sid * _RPS, _RPS)])


def _stage1_body(x_ref, w_ref, dp_ref, hs_ref, dis_ref):
    ones = jnp.ones((_NW, 1), jnp.float32)
    deg = lax.dot_general(dp_ref[...], ones, (((0,), (0,)), ((), ())),
                          preferred_element_type=jnp.float32)
    dis = lax.rsqrt(jnp.maximum(deg, 1.0))
    h = jnp.dot(x_ref[...], w_ref[...], preferred_element_type=jnp.float32)
    hs_ref[...] = h * dis
    dis_ref[...] = dis


def _stage3_body(a_ref, dis_ref, b1_ref, w_ref, hs_ref):
    dis = dis_ref[...]
    agg = jnp.concatenate([a_ref[0], a_ref[1]], axis=1)
    a = agg * dis + b1_ref[...]
    hidden = jnp.maximum(a, 0.0)
    h2 = jnp.dot(hidden, w_ref[...], preferred_element_type=jnp.float32)
    hs_ref[...] = h2 * dis


def _decoder_body(a_ref, dis_ref, b2_ref, out_ref, z_ref, z_scr):
    i = pl.program_id(0)

    @pl.when(i == 0)
    def _():
        z_scr[...] = (a_ref[0] + a_ref[1]) * dis_ref[...] + b2_ref[...]
        z_ref[...] = z_scr[pl.ds(0, _N), :]

    zb = z_scr[pl.ds(i * _BI, _BI), :]
    za = z_scr[pl.ds(0, _N), :]
    acc = lax.dot_general(zb, za, (((1,), (1,)), ((), ())),
                          preferred_element_type=jnp.float32)
    out_ref[...] = jax.nn.sigmoid(acc)


def _decoder(agg2, dis, b2row):
    return pl.pallas_call(
        _decoder_body,
        grid=(_N // _BI,),
        in_specs=[
            pl.BlockSpec((_NC, _NP, 16), lambda i: (0, 0, 0)),
            pl.BlockSpec((_NP, 1), lambda i: (0, 0)),
            pl.BlockSpec((1, 16), lambda i: (0, 0)),
        ],
        out_specs=[
            pl.BlockSpec((_BI, _N), lambda i: (i, 0)),
            pl.BlockSpec((_N, 16), lambda i: (0, 0)),
        ],
        out_shape=[jax.ShapeDtypeStruct((_N, _N), jnp.float32),
                   jax.ShapeDtypeStruct((_N, 16), jnp.float32)],
        scratch_shapes=[pltpu.VMEM((_NP, 16), jnp.float32)],
    )(agg2, dis, b2row)


def kernel(X, edge_index, W1, b1, W2, b2):
    E = edge_index.shape[1]
    pad = jnp.full((_EPAD - E,), _N, jnp.int32)
    srcp = jnp.concatenate([edge_index[0], pad])
    dstp = jnp.concatenate([edge_index[1], pad])
    Xp = jnp.pad(X, ((0, _NP - _N), (0, 0)))

    degacc = _deg_kernel(dstp)

    hs1, dis = pl.pallas_call(
        _stage1_body,
        out_shape=[jax.ShapeDtypeStruct((_NP, 64), jnp.float32),
                   jax.ShapeDtypeStruct((_NP, 1), jnp.float32)],
    )(Xp, W1, degacc)

    agg1 = _scat64_colsplit(hs1, srcp, dstp)

    hs2 = pl.pallas_call(
        _stage3_body,
        out_shape=jax.ShapeDtypeStruct((_NP, 16), jnp.float32),
    )(agg1, dis, b1.reshape(1, 64), W2)

    agg2 = _scat16(hs2, srcp, dstp)

    adj, z = _decoder(agg2, dis, b2.reshape(1, 16))
    return (adj, z, z, z)


# speedup vs baseline: 1.0544x; 1.0205x over previous
import functools

import jax
import jax.numpy as jnp
from jax import lax
from jax.experimental import pallas as pl
from jax.experimental.pallas import tpu as pltpu
from jax.experimental.pallas import tpu_sc as plsc

_N = 10000
_NP = 10112
_NC, _NS = 2, 16
_NW = _NC * _NS
_RPS = _NP // _NS
_EPAD = 163840
_EPW = _EPAD // _NW
_BLK = 128
_NBLK = _EPW // _BLK
_BI = 400

_mesh = plsc.VectorSubcoreMesh(core_axis_name="c", subcore_axis_name="s",
                               num_cores=_NC, num_subcores=_NS)
_sc_params = pltpu.CompilerParams(use_tc_tiling_on_sc=False,
                                  needs_layout_passes=False)


def _zero_rows(ref, d):
    def zrow(i, carry):
        for j in range(d // 16):
            ref[i, pl.ds(j * 16, 16)] = jnp.zeros((16,), jnp.float32)
        return carry
    lax.fori_loop(0, _RPS, zrow, 0)


def _make_gather_scatter(d, stage_table):
    scratch = [
        pltpu.VMEM((_EPW,), jnp.int32),
        pltpu.VMEM((_EPW,), jnp.int32),
        pltpu.VMEM((_BLK,), jnp.int32),
        pltpu.VMEM((_BLK, d), jnp.float32),
        pltpu.VMEM((_BLK, d), jnp.float32),
        pltpu.VMEM((_RPS, d), jnp.float32),
        pltpu.VMEM_SHARED((_NP, d), jnp.float32),
        pltpu.SemaphoreType.DMA,
        pltpu.SemaphoreType.DMA,
    ]
    if stage_table:
        scratch.insert(7, pltpu.VMEM_SHARED((_NP, d), jnp.float32))

    @functools.partial(
        pl.kernel,
        out_type=jax.ShapeDtypeStruct((_NC, _NP, d), jnp.float32),
        mesh=_mesh,
        compiler_params=_sc_params,
        scratch_types=scratch,
    )
    def scat(table, edges, out, src_v, dst_v, dst_cur, rows0, rows1,
             stage_v, acc, *rest):
        tbl_s = rest[0] if stage_table else None
        sem0, sem1 = rest[-2], rest[-1]
        cid = lax.axis_index("c")
        sid = lax.axis_index("s")
        wid = sid * _NC + cid
        base = wid * _EPW
        pltpu.sync_copy(edges.at[0, pl.ds(base, _EPW)], src_v)
        pltpu.sync_copy(edges.at[1, pl.ds(base, _EPW)], dst_v)
        if stage_table:
            pltpu.sync_copy(table.at[pl.ds(sid * _RPS, _RPS)], stage_v)
            pltpu.sync_copy(stage_v, tbl_s.at[pl.ds(sid * _RPS, _RPS)])
        _zero_rows(stage_v, d)
        pltpu.sync_copy(stage_v, acc.at[pl.ds(sid * _RPS, _RPS)])
        plsc.subcore_barrier()

        gsrc = tbl_s if stage_table else table

        def gather(blk, rbuf, sem):
            pltpu.async_copy(gsrc.at[src_v.at[pl.ds(blk * _BLK, _BLK)]],
                             rbuf, sem)

        def scatter(blk, rbuf):
            off = blk * _BLK
            for j in range(_BLK // 16):
                dst_cur[pl.ds(j * 16, 16)] = dst_v[pl.ds(off + j * 16, 16)]
            pltpu.sync_copy(rbuf, acc.at[dst_cur], add=True)

        gather(0, rows0, sem0)

        def step(i, carry):
            b0 = 2 * i
            pltpu.make_async_copy(table.at[src_v.at[pl.ds(0, _BLK)]], rows0,
                                  sem0).wait()
            gather(b0 + 1, rows1, sem1)
            scatter(b0, rows0)
            pltpu.make_async_copy(table.at[src_v.at[pl.ds(0, _BLK)]], rows1,
                                  sem1).wait()
            gather(jnp.minimum(b0 + 2, _NBLK - 1), rows0, sem0)
            scatter(b0 + 1, rows1)
            return carry

        lax.fori_loop(0, _NBLK // 2, step, 0)
        pltpu.make_async_copy(table.at[src_v.at[pl.ds(0, _BLK)]], rows0,
                              sem0).wait()
        plsc.subcore_barrier()
        pltpu.sync_copy(acc.at[pl.ds(sid * _RPS, _RPS)], stage_v)
        pltpu.sync_copy(stage_v, out.at[cid, pl.ds(sid * _RPS, _RPS)])

    return scat


@functools.partial(
    pl.kernel,
    out_type=jax.ShapeDtypeStruct((_NW, _NP), jnp.float32),
    mesh=_mesh,
    compiler_params=_sc_params,
    scratch_types=[
        pltpu.VMEM((_EPW,), jnp.int32),
        pltpu.VMEM((_NP,), jnp.float32),
    ],
)
def _deg_kernel(edges, out, dst_v, deg_v):
    cid = lax.axis_index("c")
    sid = lax.axis_index("s")
    wid = sid * _NC + cid
    pltpu.sync_copy(edges.at[1, pl.ds(wid * _EPW, _EPW)], dst_v)

    def zero(i, carry):
        deg_v[pl.ds(i * 16, 16)] = jnp.zeros((16,), jnp.float32)
        return carry
    lax.fori_loop(0, _NP // 16, zero, 0)

    ones = jnp.full((16,), 1.0, jnp.float32)

    def step(i, carry):
        idx = dst_v[pl.ds(i * 16, 16)]
        plsc.addupdate_scatter(deg_v, [idx], ones)
        return carry

    lax.fori_loop(0, _EPW // 16, step, 0)
    pltpu.sync_copy(deg_v, out.at[wid])


_scat16 = _make_gather_scatter(16, stage_table=True)

_EPT = _EPAD // _NS
_NBLKC = _EPT // _BLK


@functools.partial(
    pl.kernel,
    out_type=jax.ShapeDtypeStruct((_NC, _NP, 32), jnp.float32),
    mesh=_mesh,
    compiler_params=_sc_params,
    scratch_types=[
        pltpu.VMEM((_EPT,), jnp.int32),
        pltpu.VMEM((_EPT,), jnp.int32),
        pltpu.VMEM((_BLK,), jnp.int32),
        pltpu.VMEM((_BLK, 32), jnp.float32),
        pltpu.VMEM((_BLK, 32), jnp.float32),
        pltpu.VMEM((_RPS, 32), jnp.float32),
        pltpu.VMEM_SHARED((_NP, 32), jnp.float32),
        pltpu.VMEM_SHARED((_NP, 32), jnp.float32),
        pltpu.SemaphoreType.DMA,
        pltpu.SemaphoreType.DMA,
    ],
)
def _scat64_colsplit(table, edges, out, src_v, dst_v, dst_cur, rows0,
                     rows1, stage_v, acc, tbl_s, sem0, sem1):
    cid = lax.axis_index("c")
    sid = lax.axis_index("s")
    base = sid * _EPT
    pltpu.sync_copy(edges.at[0, pl.ds(base, _EPT)], src_v)
    pltpu.sync_copy(edges.at[1, pl.ds(base, _EPT)], dst_v)
    pltpu.sync_copy(
        table.at[pl.ds(sid * _RPS, _RPS), pl.ds(cid * 32, 32)], stage_v)
    pltpu.sync_copy(stage_v, tbl_s.at[pl.ds(sid * _RPS, _RPS)])
    _zero_rows(stage_v, 32)
    pltpu.sync_copy(stage_v, acc.at[pl.ds(sid * _RPS, _RPS)])
    plsc.subcore_barrier()

    def gather(blk, rbuf, sem):
        pltpu.async_copy(tbl_s.at[src_v.at[pl.ds(blk * _BLK, _BLK)]],
                         rbuf, sem)

    def scatter(blk, rbuf):
        off = blk * _BLK
        for j in range(_BLK // 16):
            dst_cur[pl.ds(j * 16, 16)] = dst_v[pl.ds(off + j * 16, 16)]
        pltpu.sync_copy(rbuf, acc.at[dst_cur], add=True)

    gather(0, rows0, sem0)

    def step(i, carry):
        b0 = 2 * i
        pltpu.make_async_copy(table.at[pl.ds(0, _BLK), pl.ds(0, 32)], rows0,
                              sem0).wait()
        gather(b0 + 1, rows1, sem1)
        scatter(b0, rows0)
        pltpu.make_async_copy(table.at[pl.ds(0, _BLK), pl.ds(0, 32)], rows1,
                              sem1).wait()
        gather(jnp.minimum(b0 + 2, _NBLKC - 1), rows0, sem0)
        scatter(b0 + 1, rows1)
        return carry

    lax.fori_loop(0, _NBLKC // 2, step, 0)
    pltpu.make_async_copy(table.at[pl.ds(0, _BLK), pl.ds(0, 32)], rows0,
                          sem0).wait()
    plsc.subcore_barrier()
    pltpu.sync_copy(acc.at[pl.ds(sid * _RPS, _RPS)], stage_v)
    pltpu.sync_copy(stage_v, out.at[cid, pl.ds(sid * _RPS, _RPS)])


def _stage1_body(x_ref, w_ref, dp_ref, hs_ref, dis_ref):
    ones = jnp.ones((_NW, 1), jnp.float32)
    deg = lax.dot_general(dp_ref[...], ones, (((0,), (0,)), ((), ())),
                          preferred_element_type=jnp.float32)
    dis = lax.rsqrt(jnp.maximum(deg, 1.0))
    h = jnp.dot(x_ref[...], w_ref[...], preferred_element_type=jnp.float32)
    hs_ref[...] = h * dis
    dis_ref[...] = dis


def _stage3_body(a_ref, dis_ref, b1_ref, w_ref, hs_ref):
    dis = dis_ref[...]
    agg = jnp.concatenate([a_ref[0], a_ref[1]], axis=1)
    a = agg * dis + b1_ref[...]
    hidden = jnp.maximum(a, 0.0)
    h2 = jnp.dot(hidden, w_ref[...], preferred_element_type=jnp.float32)
    hs_ref[...] = h2 * dis


def _decoder_body(a_ref, dis_ref, b2_ref, out_ref, z_ref, z_scr):
    i = pl.program_id(0)

    @pl.when(i == 0)
    def _():
        z_scr[...] = (a_ref[0] + a_ref[1]) * dis_ref[...] + b2_ref[...]
        z_ref[...] = z_scr[pl.ds(0, _N), :]

    zb = z_scr[pl.ds(i * _BI, _BI), :]
    za = z_scr[pl.ds(0, _N), :]
    acc = lax.dot_general(zb, za, (((1,), (1,)), ((), ())),
                          preferred_element_type=jnp.float32)
    out_ref[...] = jax.nn.sigmoid(acc)


def _decoder(agg2, dis, b2row):
    return pl.pallas_call(
        _decoder_body,
        grid=(_N // _BI,),
        in_specs=[
            pl.BlockSpec((_NC, _NP, 16), lambda i: (0, 0, 0)),
            pl.BlockSpec((_NP, 1), lambda i: (0, 0)),
            pl.BlockSpec((1, 16), lambda i: (0, 0)),
        ],
        out_specs=[
            pl.BlockSpec((_BI, _N), lambda i: (i, 0)),
            pl.BlockSpec((_N, 16), lambda i: (0, 0)),
        ],
        out_shape=[jax.ShapeDtypeStruct((_N, _N), jnp.float32),
                   jax.ShapeDtypeStruct((_N, 16), jnp.float32)],
        scratch_shapes=[pltpu.VMEM((_NP, 16), jnp.float32)],
    )(agg2, dis, b2row)


def kernel(X, edge_index, W1, b1, W2, b2):
    E = edge_index.shape[1]
    edges = jnp.pad(edge_index, ((0, 0), (0, _EPAD - E)), constant_values=_N)
    Xp = jnp.pad(X, ((0, _NP - _N), (0, 0)))

    degacc = _deg_kernel(edges)

    hs1, dis = pl.pallas_call(
        _stage1_body,
        out_shape=[jax.ShapeDtypeStruct((_NP, 64), jnp.float32),
                   jax.ShapeDtypeStruct((_NP, 1), jnp.float32)],
    )(Xp, W1, degacc)

    agg1 = _scat64_colsplit(hs1, edges)

    hs2 = pl.pallas_call(
        _stage3_body,
        out_shape=jax.ShapeDtypeStruct((_NP, 16), jnp.float32),
    )(agg1, dis, b1.reshape(1, 64), W2)

    agg2 = _scat16(hs2, edges)

    adj, z = _decoder(agg2, dis, b2.reshape(1, 16))
    return (adj, z, z, z)
